# Initial kernel scaffold; baseline (speedup 1.0000x reference)
import jax
import jax.numpy as jnp
from jax.experimental import pallas as pl


def kernel(x, ei_to, ei_parentof, ei_childof, ei_canprecede, ei_canfollow, ei_peerof, Ws1, Wd1, atts1, attd1, b1, Ws2, Wd2, atts2, attd2, b2):
    raise NotImplementedError("write your pallas kernel here")


# speedup vs baseline: 2.6577x; 2.6577x over previous
import jax
import jax.numpy as jnp
from jax.experimental import pallas as pl

_BM = 1000


def _mm_kernel(x_ref, w_ref, wa_ref, o_ref, oa_ref):
    x = x_ref[...]
    o_ref[...] = jnp.dot(x, w_ref[...], preferred_element_type=jnp.float32)
    oa_ref[...] = jnp.dot(x, wa_ref[...], preferred_element_type=jnp.float32)


def _mm_relu_kernel(x_ref, b_ref, w_ref, wa_ref, o_ref, oa_ref):
    x = jnp.maximum(x_ref[...] + b_ref[...], 0.0)
    o_ref[...] = jnp.dot(x, w_ref[...], preferred_element_type=jnp.float32)
    oa_ref[...] = jnp.dot(x, wa_ref[...], preferred_element_type=jnp.float32)


def _dense(x, w, wa, bias=None):
    n, d = x.shape
    k = w.shape[1]
    bm = _BM
    n_pad = (-n) % bm
    if n_pad:
        x = jnp.pad(x, ((0, n_pad), (0, 0)))
    npad = n + n_pad
    grid = (npad // bm,)
    wa_p = jnp.pad(wa, ((0, 0), (0, 128 - wa.shape[1])))
    out_shape = [
        jax.ShapeDtypeStruct((npad, k), jnp.float32),
        jax.ShapeDtypeStruct((npad, 128), jnp.float32),
    ]
    out_specs = [
        pl.BlockSpec((bm, k), lambda i: (i, 0)),
        pl.BlockSpec((bm, 128), lambda i: (i, 0)),
    ]
    if bias is None:
        hs, aa = pl.pallas_call(
            _mm_kernel,
            grid=grid,
            in_specs=[
                pl.BlockSpec((bm, d), lambda i: (i, 0)),
                pl.BlockSpec((d, k), lambda i: (0, 0)),
                pl.BlockSpec((d, 128), lambda i: (0, 0)),
            ],
            out_specs=out_specs,
            out_shape=out_shape,
        )(x, w, wa_p)
    else:
        hs, aa = pl.pallas_call(
            _mm_relu_kernel,
            grid=grid,
            in_specs=[
                pl.BlockSpec((bm, d), lambda i: (i, 0)),
                pl.BlockSpec((1, d), lambda i: (0, 0)),
                pl.BlockSpec((d, k), lambda i: (0, 0)),
                pl.BlockSpec((d, 128), lambda i: (0, 0)),
            ],
            out_specs=out_specs,
            out_shape=out_shape,
        )(x, bias.reshape(1, d), w, wa_p)
    if n_pad:
        hs = hs[:n]
        aa = aa[:n]
    return hs, aa[:, :4]


def _edges(hs, a_s, a_d, eis):
    n = hs.shape[0]
    nrel = len(eis)
    hc = hs.shape[1]
    h = 2
    loop = jnp.arange(n, dtype=eis[0].dtype)
    dsts = [jnp.concatenate([ei[1], loop]) for ei in eis]
    src = jnp.concatenate([jnp.concatenate([ei[0], loop]) for ei in eis])
    seg = jnp.concatenate([d_ + (r * n) for r, d_ in enumerate(dsts)])
    dst = jnp.concatenate(dsts)
    alpha = a_s[src] + a_d[dst]
    alpha = jnp.where(alpha > 0, alpha, 0.2 * alpha)
    amax = jax.ops.segment_max(alpha, seg, num_segments=nrel * n)
    ex = jnp.exp(alpha - amax[seg])
    den = jax.ops.segment_sum(ex, seg, num_segments=nrel * n)
    coef = ex / den[seg]
    msg = hs[src].reshape(-1, h, hc // h) * coef[:, :, None]
    out = jax.ops.segment_sum(msg.reshape(-1, hc), seg, num_segments=nrel * n)
    return out.reshape(nrel, n, hc).sum(0)


def kernel(x, ei_to, ei_parentof, ei_childof, ei_canprecede, ei_canfollow,
           ei_peerof, Ws1, Wd1, atts1, attd1, b1, Ws2, Wd2, atts2, attd2, b2):
    eis = [ei_to, ei_parentof, ei_childof, ei_canprecede, ei_canfollow, ei_peerof]
    nrel = float(len(eis))
    h = 2

    def watt(w, att):
        d = w.shape[0]
        c = att.shape[1]
        return (w.reshape(d, h, c) * att[None]).sum(-1)

    wa1 = jnp.concatenate([watt(Ws1, atts1), watt(Wd1, attd1)], axis=1)
    wa2 = jnp.concatenate([watt(Ws2, atts2), watt(Wd2, attd2)], axis=1)

    hs1, aa1 = _dense(x, Ws1, wa1)
    h1 = _edges(hs1, aa1[:, :2], aa1[:, 2:4], eis)
    hs2, aa2 = _dense(h1, Ws2, wa2, bias=nrel * b1)
    out = _edges(hs2, aa2[:, :2], aa2[:, 2:4], eis)
    return out + nrel * b2


# speedup vs baseline: 3.8885x; 1.4631x over previous
import jax
import jax.numpy as jnp
from jax.experimental import pallas as pl

_BM = 1000


def _mm_kernel(x_ref, w_ref, wa_ref, o_ref, oa_ref):
    x = x_ref[...]
    o_ref[...] = jnp.dot(x, w_ref[...], preferred_element_type=jnp.float32)
    oa_ref[...] = jnp.dot(x, wa_ref[...], preferred_element_type=jnp.float32)


def _mm_relu_kernel(x_ref, b_ref, w_ref, wa_ref, o_ref, oa_ref):
    x = jnp.maximum(x_ref[...] + b_ref[...], 0.0)
    o_ref[...] = jnp.dot(x, w_ref[...], preferred_element_type=jnp.float32)
    oa_ref[...] = jnp.dot(x, wa_ref[...], preferred_element_type=jnp.float32)


def _dense(x, w, wa, bias=None):
    n, d = x.shape
    k = w.shape[1]
    bm = _BM
    n_pad = (-n) % bm
    if n_pad:
        x = jnp.pad(x, ((0, n_pad), (0, 0)))
    npad = n + n_pad
    grid = (npad // bm,)
    wa_p = jnp.pad(wa, ((0, 0), (0, 128 - wa.shape[1])))
    out_shape = [
        jax.ShapeDtypeStruct((npad, k), jnp.float32),
        jax.ShapeDtypeStruct((npad, 128), jnp.float32),
    ]
    out_specs = [
        pl.BlockSpec((bm, k), lambda i: (i, 0)),
        pl.BlockSpec((bm, 128), lambda i: (i, 0)),
    ]
    if bias is None:
        hs, aa = pl.pallas_call(
            _mm_kernel,
            grid=grid,
            in_specs=[
                pl.BlockSpec((bm, d), lambda i: (i, 0)),
                pl.BlockSpec((d, k), lambda i: (0, 0)),
                pl.BlockSpec((d, 128), lambda i: (0, 0)),
            ],
            out_specs=out_specs,
            out_shape=out_shape,
        )(x, w, wa_p)
    else:
        hs, aa = pl.pallas_call(
            _mm_relu_kernel,
            grid=grid,
            in_specs=[
                pl.BlockSpec((bm, d), lambda i: (i, 0)),
                pl.BlockSpec((1, d), lambda i: (0, 0)),
                pl.BlockSpec((d, k), lambda i: (0, 0)),
                pl.BlockSpec((d, 128), lambda i: (0, 0)),
            ],
            out_specs=out_specs,
            out_shape=out_shape,
        )(x, bias.reshape(1, d), w, wa_p)
    if n_pad:
        hs = hs[:n]
        aa = aa[:n]
    return hs, aa[:, :4]


def _edges(hs, a_s, a_d, eis):
    n = hs.shape[0]
    nrel = len(eis)
    hc = hs.shape[1]
    h = 2
    c = hc // h
    src = jnp.concatenate([ei[0] for ei in eis])
    dst = jnp.concatenate([ei[1] for ei in eis])
    seg6 = jnp.concatenate([ei[1] + (r * n) for r, ei in enumerate(eis)])
    alpha = a_s[src] + a_d[dst]
    alpha = jnp.where(alpha > 0, alpha, 0.2 * alpha)
    aself = a_s + a_d
    aself = jnp.where(aself > 0, aself, 0.2 * aself)
    amax_e = jax.ops.segment_max(alpha, seg6, num_segments=nrel * n)
    amax = jnp.maximum(amax_e.reshape(nrel, n, h), aself[None])
    amax_f = amax.reshape(nrel * n, h)
    ex = jnp.exp(alpha - amax_f[seg6])
    exself = jnp.exp(aself[None] - amax)
    den = jax.ops.segment_sum(ex, seg6, num_segments=nrel * n)
    den = den.reshape(nrel, n, h) + exself
    coef = ex / den.reshape(nrel * n, h)[seg6]
    selfcoef = (exself / den).sum(0)
    msg = hs[src].reshape(-1, h, c) * coef[:, :, None]
    out = jax.ops.segment_sum(msg.reshape(-1, hc), dst, num_segments=n)
    return out + (hs.reshape(n, h, c) * selfcoef[:, :, None]).reshape(n, hc)


def kernel(x, ei_to, ei_parentof, ei_childof, ei_canprecede, ei_canfollow,
           ei_peerof, Ws1, Wd1, atts1, attd1, b1, Ws2, Wd2, atts2, attd2, b2):
    eis = [ei_to, ei_parentof, ei_childof, ei_canprecede, ei_canfollow, ei_peerof]
    nrel = float(len(eis))
    h = 2

    def watt(w, att):
        d = w.shape[0]
        c = att.shape[1]
        return (w.reshape(d, h, c) * att[None]).sum(-1)

    wa1 = jnp.concatenate([watt(Ws1, atts1), watt(Wd1, attd1)], axis=1)
    wa2 = jnp.concatenate([watt(Ws2, atts2), watt(Wd2, attd2)], axis=1)

    hs1, aa1 = _dense(x, Ws1, wa1)
    h1 = _edges(hs1, aa1[:, :2], aa1[:, 2:4], eis)
    hs2, aa2 = _dense(h1, Ws2, wa2, bias=nrel * b1)
    out = _edges(hs2, aa2[:, :2], aa2[:, 2:4], eis)
    return out + nrel * b2


# speedup vs baseline: 4.0162x; 1.0328x over previous
import jax
import jax.numpy as jnp
from jax.experimental import pallas as pl

_BM = 1000


def _mm_kernel(x_ref, w_ref, wa_ref, o_ref, oa_ref):
    x = x_ref[...]
    o_ref[...] = jnp.dot(x, w_ref[...], preferred_element_type=jnp.float32)
    oa_ref[...] = jnp.dot(x, wa_ref[...], preferred_element_type=jnp.float32)


def _mm_relu_kernel(x_ref, b_ref, w_ref, wa_ref, o_ref, oa_ref):
    x = jnp.maximum(x_ref[...] + b_ref[...], 0.0)
    o_ref[...] = jnp.dot(x, w_ref[...], preferred_element_type=jnp.float32)
    oa_ref[...] = jnp.dot(x, wa_ref[...], preferred_element_type=jnp.float32)


def _dense(x, w, wa, bias=None):
    n, d = x.shape
    k = w.shape[1]
    bm = _BM
    n_pad = (-n) % bm
    if n_pad:
        x = jnp.pad(x, ((0, n_pad), (0, 0)))
    npad = n + n_pad
    grid = (npad // bm,)
    wa_p = jnp.pad(wa, ((0, 0), (0, 128 - wa.shape[1])))
    out_shape = [
        jax.ShapeDtypeStruct((npad, k), jnp.float32),
        jax.ShapeDtypeStruct((npad, 128), jnp.float32),
    ]
    out_specs = [
        pl.BlockSpec((bm, k), lambda i: (i, 0)),
        pl.BlockSpec((bm, 128), lambda i: (i, 0)),
    ]
    if bias is None:
        hs, aa = pl.pallas_call(
            _mm_kernel,
            grid=grid,
            in_specs=[
                pl.BlockSpec((bm, d), lambda i: (i, 0)),
                pl.BlockSpec((d, k), lambda i: (0, 0)),
                pl.BlockSpec((d, 128), lambda i: (0, 0)),
            ],
            out_specs=out_specs,
            out_shape=out_shape,
        )(x, w, wa_p)
    else:
        hs, aa = pl.pallas_call(
            _mm_relu_kernel,
            grid=grid,
            in_specs=[
                pl.BlockSpec((bm, d), lambda i: (i, 0)),
                pl.BlockSpec((1, d), lambda i: (0, 0)),
                pl.BlockSpec((d, k), lambda i: (0, 0)),
                pl.BlockSpec((d, 128), lambda i: (0, 0)),
            ],
            out_specs=out_specs,
            out_shape=out_shape,
        )(x, bias.reshape(1, d), w, wa_p)
    if n_pad:
        hs = hs[:n]
        aa = aa[:n]
    return hs, aa[:, :4]


def _edges(hs, a_s, a_d, eis):
    n = hs.shape[0]
    src, dst, seg6, nrel = eis
    hc = hs.shape[1]
    h = 2
    c = hc // h
    alpha = a_s[src] + a_d[dst]
    alpha = jnp.where(alpha > 0, alpha, 0.2 * alpha)
    aself = a_s + a_d
    aself = jnp.where(aself > 0, aself, 0.2 * aself)
    amax_e = jax.ops.segment_max(alpha, seg6, num_segments=nrel * n)
    amax = jnp.maximum(amax_e.reshape(nrel, n, h), aself[None])
    amax_f = amax.reshape(nrel * n, h)
    ex = jnp.exp(alpha - amax_f[seg6])
    exself = jnp.exp(aself[None] - amax)
    den = jax.ops.segment_sum(ex, seg6, num_segments=nrel * n)
    den = den.reshape(nrel, n, h) + exself
    coef = ex / den.reshape(nrel * n, h)[seg6]
    selfcoef = (exself / den).sum(0)
    msg = hs[src].reshape(-1, h, c) * coef[:, :, None]
    out = jax.ops.segment_sum(msg.reshape(-1, hc), dst, num_segments=n,
                              indices_are_sorted=True)
    return out + (hs.reshape(n, h, c) * selfcoef[:, :, None]).reshape(n, hc)


def kernel(x, ei_to, ei_parentof, ei_childof, ei_canprecede, ei_canfollow,
           ei_peerof, Ws1, Wd1, atts1, attd1, b1, Ws2, Wd2, atts2, attd2, b2):
    ei_list = [ei_to, ei_parentof, ei_childof, ei_canprecede, ei_canfollow,
               ei_peerof]
    nrel = len(ei_list)
    n = x.shape[0]
    h = 2

    src = jnp.concatenate([ei[0] for ei in ei_list])
    dst = jnp.concatenate([ei[1] for ei in ei_list])
    seg6 = jnp.concatenate([ei[1] + (r * n) for r, ei in enumerate(ei_list)])
    perm = jnp.argsort(dst)
    eis = (src[perm], dst[perm], seg6[perm], nrel)

    def watt(w, att):
        d = w.shape[0]
        c = att.shape[1]
        return (w.reshape(d, h, c) * att[None]).sum(-1)

    wa1 = jnp.concatenate([watt(Ws1, atts1), watt(Wd1, attd1)], axis=1)
    wa2 = jnp.concatenate([watt(Ws2, atts2), watt(Wd2, attd2)], axis=1)

    hs1, aa1 = _dense(x, Ws1, wa1)
    h1 = _edges(hs1, aa1[:, :2], aa1[:, 2:4], eis)
    hs2, aa2 = _dense(h1, Ws2, wa2, bias=nrel * b1)
    out = _edges(hs2, aa2[:, :2], aa2[:, 2:4], eis)
    return out + nrel * b2
